# baseline (speedup 1.0000x reference)
import functools

import jax
import jax.numpy as jnp
from jax import lax
from jax.experimental import pallas as pl
from jax.experimental.pallas import tpu as pltpu
from jax.experimental.pallas import tpu_sc as plsc

BATCH = 4096
HIST = 50
EMBED_DIM = 64
N = BATCH * HIST

_info = plsc.get_sparse_core_info()
NUM_CORES = _info.num_cores
NUM_SUBCORES = _info.num_subcores
NW = NUM_CORES * NUM_SUBCORES

KB = 8
CL = KB * HIST
NC = BATCH // NW // KB
G = 16
NGRP = CL // G

_mesh = plsc.VectorSubcoreMesh(core_axis_name="c", subcore_axis_name="s")


@functools.partial(
    pl.kernel,
    mesh=_mesh,
    out_type=jax.ShapeDtypeStruct((BATCH, HIST, EMBED_DIM), jnp.float32),
    scratch_types=[
        pltpu.VMEM((NC, CL), jnp.int32),
        pltpu.VMEM((2, KB, HIST, EMBED_DIM), jnp.float32),
        pltpu.SemaphoreType.DMA,
        pltpu.SemaphoreType.DMA,
        pltpu.SemaphoreType.DMA,
    ],
    compiler_params=pltpu.CompilerParams(use_tc_tiling_on_sc=True),
)
def _gather_kernel(idx_hbm, table_hbm, out_hbm, idx_v, stage_v,
                   sem_t0, sem_t1, sem_o):
    wid = lax.axis_index("s") * NUM_CORES + lax.axis_index("c")
    pltpu.sync_copy(idx_hbm.at[wid], idx_v)
    sems = (sem_t0, sem_t1)

    def fire(j, g, tb, sb):
        v = idx_v[j, pl.ds(g * G, G)]
        for l in range(G):
            i = g * G + l
            bb = (i * 5243) >> 18
            hh = i - bb * 50
            r = v[l]
            pltpu.async_copy(table_hbm.at[r >> 3, r & 7],
                             stage_v.at[sb, bb, hh], sems[tb])

    def drain(tb, sb):
        pltpu.make_async_copy(out_hbm.at[0, pl.ds(0, 16), :],
                              stage_v.at[sb, 0, pl.ds(0, 16)],
                              sems[tb]).wait()

    def chunk(j, carry):
        sb = j % 2

        @pl.when(j >= 2)
        def _():
            b0 = (wid * NC + j - 2) * KB
            prev = out_hbm.at[pl.ds(b0, KB)]
            pltpu.make_async_copy(stage_v.at[sb], prev, sem_o).wait()

        for g in range(NGRP):
            fire(j, g, g % 2, sb)
            if g >= 1:
                drain((g - 1) % 2, sb)
        drain((NGRP - 1) % 2, sb)

        b0 = (wid * NC + j) * KB
        pltpu.async_copy(stage_v.at[sb], out_hbm.at[pl.ds(b0, KB)], sem_o)
        return carry

    lax.fori_loop(0, NC, chunk, 0)
    for j in (NC - 2, NC - 1):
        b0 = (wid * NC + j) * KB
        pltpu.make_async_copy(stage_v.at[j % 2],
                              out_hbm.at[pl.ds(b0, KB)], sem_o).wait()


def kernel(x, table):
    t3 = table.reshape(125000, 8, EMBED_DIM)
    idx = x.reshape(NW, NC, CL).astype(jnp.int32)
    return _gather_kernel(idx, t3)


# speedup vs baseline: 1.0096x; 1.0096x over previous
import functools

import jax
import jax.numpy as jnp
from jax import lax
from jax.experimental import pallas as pl
from jax.experimental.pallas import tpu as pltpu
from jax.experimental.pallas import tpu_sc as plsc

BATCH = 4096
HIST = 50
EMBED_DIM = 64
N = BATCH * HIST

_info = plsc.get_sparse_core_info()
NUM_CORES = _info.num_cores
NUM_SUBCORES = _info.num_subcores
NW = NUM_CORES * NUM_SUBCORES

KB = 8
CL = KB * HIST
NC = BATCH // NW // KB
G = 16
NGRP = CL // G

_mesh = plsc.VectorSubcoreMesh(core_axis_name="c", subcore_axis_name="s")


@functools.partial(
    pl.kernel,
    mesh=_mesh,
    out_type=jax.ShapeDtypeStruct((BATCH, HIST, EMBED_DIM), jnp.float32),
    scratch_types=[
        pltpu.VMEM((NC, CL), jnp.int32),
        pltpu.VMEM((2, KB, HIST, EMBED_DIM), jnp.float32),
        pltpu.SemaphoreType.DMA,
        pltpu.SemaphoreType.DMA,
        pltpu.SemaphoreType.DMA,
    ],
    compiler_params=pltpu.CompilerParams(use_tc_tiling_on_sc=True),
)
def _gather_kernel(idx_hbm, table_hbm, out_hbm, idx_v, stage_v,
                   sem_t0, sem_t1, sem_o):
    wid = lax.axis_index("s") * NUM_CORES + lax.axis_index("c")
    pltpu.sync_copy(idx_hbm.at[wid], idx_v)
    sems = (sem_t0, sem_t1)

    def fire(j, g, tb, sb):
        v = idx_v[j, pl.ds(g * G, G)]
        for l in range(G):
            i = g * G + l
            bb = (i * 5243) >> 18
            hh = i - bb * 50
            r = v[l]
            pltpu.async_copy(table_hbm.at[r >> 3, r & 7],
                             stage_v.at[sb, bb, hh], sems[tb])

    def drain(tb, sb):
        pltpu.make_async_copy(out_hbm.at[0, pl.ds(0, 16), :],
                              stage_v.at[sb, 0, pl.ds(0, 16)],
                              sems[tb]).wait()

    def chunk(j, carry):
        sb = j % 2

        @pl.when(j >= 2)
        def _():
            b0 = (wid * NC + j - 2) * KB
            prev = out_hbm.at[pl.ds(b0, KB)]
            pltpu.make_async_copy(stage_v.at[sb], prev, sem_o).wait()

        fire(j, 0, 0, sb)

        def pair(p, c2):
            for tb01 in range(2):
                g = p * 2 + tb01
                fire(j, g + 1, (tb01 + 1) % 2, sb)
                drain(tb01, sb)
            return c2

        lax.fori_loop(0, (NGRP - 1) // 2, pair, 0)
        drain(0, sb)

        b0 = (wid * NC + j) * KB
        pltpu.async_copy(stage_v.at[sb], out_hbm.at[pl.ds(b0, KB)], sem_o)
        return carry

    lax.fori_loop(0, NC, chunk, 0)
    for j in (NC - 2, NC - 1):
        b0 = (wid * NC + j) * KB
        pltpu.make_async_copy(stage_v.at[j % 2],
                              out_hbm.at[pl.ds(b0, KB)], sem_o).wait()


def kernel(x, table):
    t3 = table.reshape(125000, 8, EMBED_DIM)
    idx = x.reshape(NW, NC, CL).astype(jnp.int32)
    return _gather_kernel(idx, t3)
